# baseline (speedup 1.0000x reference)
import jax
import jax.numpy as jnp
from jax import lax
from jax.experimental import pallas as pl
from jax.experimental.pallas import tpu as pltpu
from jax.experimental.pallas import tpu_sc as plsc

N = 50000
E = 800000
G = 64


NBUCK = 8
BSZ = 6272
B_ACC = 6400
BZR = 400
BPAD = 16384
TOTBLK = 7274
TOT_CAP = TOTBLK * 128

BN = 2000
NB = N // BN

_f32 = jnp.float32


def _sc_spmm(g, srcb, dstb, boff_arr, bp_arr):
    mesh = plsc.VectorSubcoreMesh(core_axis_name="c", subcore_axis_name="s")

    def body(g_hbm, srcs, dsts, boff_h, bp_h, out,
             src_v, dst_v, b0, b1, b2, b3, zidx, acc, boff_s, bp_s,
             s0, s1, s2, s3):
        cid = lax.axis_index("c")
        sid = lax.axis_index("s")
        base = lax.broadcasted_iota(jnp.int32, (16,), 0)
        bufs = [b0, b1, b2, b3]
        sems = [s0, s1, s2, s3]

        pltpu.sync_copy(boff_h, boff_s)
        pltpu.sync_copy(bp_h, bp_s)

        for r in range(4):
            for k in range(8):
                v = base + (sid * BZR + r * 128 + k * 16)
                zidx[r, k * 16:(k + 1) * 16] = jnp.minimum(v, B_ACC - 1)

        zval = jnp.zeros((16,), _f32)

        def zfill(r, _):
            for k in range(8):
                b0[r, k * 16:(k + 1) * 16] = zval
            return 0

        for j in range(NBUCK // 2):
            bucket = 2 * j + cid
            boff = boff_s[pl.ds(bucket, 16)][0]
            nc = bp_s[pl.ds(bucket, 16)][0]
            if j > 0:
                plsc.subcore_barrier()
            lax.fori_loop(0, 128, zfill, 0)
            for r in range(4):
                pltpu.sync_copy(b0, acc.at[zidx.at[r]])
            tstart = pl.multiple_of(boff + sid * nc * 8, 8)
            plsc.subcore_barrier()

            def chunk(m, _):
                moff = pl.multiple_of(tstart + m * 8, 8)
                pltpu.sync_copy(srcs.at[pl.ds(moff, 8)], src_v)
                pltpu.sync_copy(dsts.at[pl.ds(moff, 8)], dst_v)
                descs = []
                for t in range(8):
                    if t >= 4:
                        descs[t - 4].wait()
                        pltpu.sync_copy(bufs[t % 4],
                                        acc.at[dst_v.at[t - 4]], add=True)
                    descs.append(pltpu.async_copy(
                        g_hbm.at[src_v.at[t]], bufs[t % 4], sems[t % 4]))
                for t in range(4, 8):
                    descs[t].wait()
                    pltpu.sync_copy(bufs[t % 4],
                                    acc.at[dst_v.at[t]], add=True)
                return 0

            lax.fori_loop(0, nc, chunk, 0)
            plsc.subcore_barrier()
            rows_p = jnp.minimum(BSZ, N - bucket * BSZ)
            for c in range(4):
                wstart = pl.multiple_of(
                    jnp.minimum(sid * BZR + c * 128, rows_p - 128), 8)
                gstart = pl.multiple_of(bucket * BSZ + wstart, 8)
                pltpu.sync_copy(acc.at[pl.ds(wstart, 128)], b0)
                pltpu.sync_copy(b0, out.at[pl.ds(gstart, 128)])

    fn = pl.kernel(
        body,
        out_type=jax.ShapeDtypeStruct((N, 128), _f32),
        mesh=mesh,
        scratch_types=[
            pltpu.VMEM((8, 128), jnp.int32),
            pltpu.VMEM((8, 128), jnp.int32),
            pltpu.VMEM((128, 128), _f32),
            pltpu.VMEM((128, 128), _f32),
            pltpu.VMEM((128, 128), _f32),
            pltpu.VMEM((128, 128), _f32),
            pltpu.VMEM((4, 128), jnp.int32),
            pltpu.VMEM_SHARED((B_ACC, 128), _f32),
            pltpu.VMEM((32,), jnp.int32),
            pltpu.VMEM((32,), jnp.int32),
            pltpu.SemaphoreType.DMA,
            pltpu.SemaphoreType.DMA,
            pltpu.SemaphoreType.DMA,
            pltpu.SemaphoreType.DMA,
        ],
    )
    return fn(g, srcb, dstb, boff_arr, bp_arr)


def _deg_finalize(degF, cc_arr):
    def body(deg_ref, cc_ref, dinv_ref, post_ref):
        d = deg_ref[:, 0:1]
        dinv = jnp.where(d > 0, lax.rsqrt(jnp.maximum(d, 1.0)), 0.0)
        dinv_ref[...] = dinv
        post_ref[...] = (-cc_ref[0, 0]) * dinv

    return pl.pallas_call(
        body,
        grid=(NB,),
        in_specs=[
            pl.BlockSpec((BN, 128), lambda i: (i, 0)),
            pl.BlockSpec(memory_space=pltpu.SMEM),
        ],
        out_specs=[
            pl.BlockSpec((BN, 1), lambda i: (i, 0)),
            pl.BlockSpec((BN, 1), lambda i: (i, 0)),
        ],
        out_shape=[
            jax.ShapeDtypeStruct((N, 1), _f32),
            jax.ShapeDtypeStruct((N, 1), _f32),
        ],
    )(degF, cc_arr)


def _layer_init(z, b, W0, dinv, first):
    Fz = z.shape[1]
    Fout = W0.shape[1]

    def body(*refs):
        if first:
            z_ref, W_ref, dinv_ref, h_ref, out_ref, g_ref = refs
        else:
            z_ref, b_ref, W_ref, dinv_ref, h_ref, out_ref, g_ref = refs
        h = z_ref[...]
        if not first:
            h = jnp.maximum(h + b_ref[...][None, :], 0.0)
        h_ref[...] = h
        out_ref[...] = jnp.dot(h, W_ref[...], preferred_element_type=_f32)
        gv = dinv_ref[...] * h
        if Fz < 128:
            gv = jnp.concatenate(
                [gv, jnp.zeros((BN, 128 - Fz), _f32)], axis=1)
        g_ref[...] = gv

    in_specs = [pl.BlockSpec((BN, Fz), lambda i: (i, 0))]
    args = [z]
    if not first:
        in_specs.append(pl.BlockSpec((Fz,), lambda i: (0,)))
        args.append(b)
    in_specs += [
        pl.BlockSpec((Fz, Fout), lambda i: (0, 0)),
        pl.BlockSpec((BN, 1), lambda i: (i, 0)),
    ]
    args += [W0, dinv]
    return pl.pallas_call(
        body,
        grid=(NB,),
        in_specs=in_specs,
        out_specs=[
            pl.BlockSpec((BN, Fz), lambda i: (i, 0)),
            pl.BlockSpec((BN, Fout), lambda i: (i, 0)),
            pl.BlockSpec((BN, 128), lambda i: (i, 0)),
        ],
        out_shape=[
            jax.ShapeDtypeStruct((N, Fz), _f32),
            jax.ShapeDtypeStruct((N, Fout), _f32),
            jax.ShapeDtypeStruct((N, 128), _f32),
        ],
    )(*args)


def _cheb_step(P, TxA, TxB, post, dinv, Wk, out_in, diag_arr,
               alpha, beta, emit_g):
    F = TxA.shape[1]
    Fout = Wk.shape[1]

    def body(*refs):
        (P_ref, TxA_ref, TxB_ref, post_ref, dinv_ref, W_ref, oin_ref,
         diag_ref) = refs[:8]
        outs = refs[8:]
        if emit_g:
            TxN_ref, oout_ref, g_ref = outs
        else:
            TxN_ref, oout_ref = outs
        dg = diag_ref[0, 0]
        S = P_ref[...][:, :F]
        t = alpha * (post_ref[...] * S + dg * TxA_ref[...])
        if beta:
            t = t + beta * TxB_ref[...]
        TxN_ref[...] = t
        if emit_g:
            gv = dinv_ref[...] * t
            if F < 128:
                gv = jnp.concatenate(
                    [gv, jnp.zeros((BN, 128 - F), _f32)], axis=1)
            g_ref[...] = gv
        oout_ref[...] = oin_ref[...] + jnp.dot(
            t, W_ref[...], preferred_element_type=_f32)

    out_specs = [
        pl.BlockSpec((BN, F), lambda i: (i, 0)),
        pl.BlockSpec((BN, Fout), lambda i: (i, 0)),
    ]
    out_shape = [
        jax.ShapeDtypeStruct((N, F), _f32),
        jax.ShapeDtypeStruct((N, Fout), _f32),
    ]
    if emit_g:
        out_specs.append(pl.BlockSpec((BN, 128), lambda i: (i, 0)))
        out_shape.append(jax.ShapeDtypeStruct((N, 128), _f32))
    res = pl.pallas_call(
        body,
        grid=(NB,),
        in_specs=[
            pl.BlockSpec((BN, 128), lambda i: (i, 0)),
            pl.BlockSpec((BN, F), lambda i: (i, 0)),
            pl.BlockSpec((BN, F), lambda i: (i, 0)),
            pl.BlockSpec((BN, 1), lambda i: (i, 0)),
            pl.BlockSpec((BN, 1), lambda i: (i, 0)),
            pl.BlockSpec((F, Fout), lambda i: (0, 0)),
            pl.BlockSpec((BN, Fout), lambda i: (i, 0)),
            pl.BlockSpec(memory_space=pltpu.SMEM),
        ],
        out_specs=out_specs,
        out_shape=out_shape,
    )(P, TxA, TxB, post, dinv, Wk, out_in, diag_arr)
    if emit_g:
        return res
    return res[0], res[1], None


def _pool_mlp(out3, b3, batch, fc1_w, fc1_b, fc2_w, fc2_b):
    def body(h_ref, b3_ref, bat_ref, w1_ref, b1_ref, w2_ref, b2_ref,
             out_ref, sums, cnts):
        i = pl.program_id(0)

        @pl.when(i == 0)
        def _():
            sums[...] = jnp.zeros_like(sums)
            cnts[...] = jnp.zeros_like(cnts)

        h = jnp.maximum(h_ref[...] + b3_ref[...][None, :], 0.0)
        seg = lax.broadcasted_iota(jnp.int32, (G, BN), 0)
        onehot = jnp.where(seg == bat_ref[0], 1.0, 0.0)
        sums[...] += jnp.dot(onehot, h, preferred_element_type=_f32)
        cnts[...] += jnp.sum(onehot, axis=1, keepdims=True)

        @pl.when(i == NB - 1)
        def _():
            pooled = sums[...] / jnp.maximum(cnts[...], 1.0)
            h2 = jnp.maximum(
                jnp.dot(pooled, w1_ref[...], preferred_element_type=_f32)
                + b1_ref[...][None, :], 0.0)
            logits = jnp.dot(h2, w2_ref[...], preferred_element_type=_f32) \
                + b2_ref[...][None, :]
            m = jnp.max(logits, axis=1, keepdims=True)
            lse = jnp.log(jnp.sum(jnp.exp(logits - m), axis=1,
                                  keepdims=True)) + m
            out_ref[...] = logits - lse

    return pl.pallas_call(
        body,
        grid=(NB,),
        in_specs=[
            pl.BlockSpec((BN, 128), lambda i: (i, 0)),
            pl.BlockSpec((128,), lambda i: (0,)),
            pl.BlockSpec((1, 1, BN), lambda i: (i, 0, 0)),
            pl.BlockSpec((128, 32), lambda i: (0, 0)),
            pl.BlockSpec((32,), lambda i: (0,)),
            pl.BlockSpec((32, 10), lambda i: (0, 0)),
            pl.BlockSpec((10,), lambda i: (0,)),
        ],
        out_specs=pl.BlockSpec((G, 10), lambda i: (0, 0)),
        out_shape=jax.ShapeDtypeStruct((G, 10), _f32),
        scratch_shapes=[
            pltpu.VMEM((G, 128), _f32),
            pltpu.VMEM((G, 128), _f32),
        ],
    )(out3, b3, batch.reshape(NB, 1, BN), fc1_w, fc1_b, fc2_w, fc2_b)


def _cheb_layer(h0, bias, W, dinv, post, diag_arr, first, spmm):
    K = W.shape[0]
    TxA, out, g = _layer_init(h0, bias, W[0], dinv, first)
    TxB = TxA
    for k in range(1, K):
        P = spmm(g)
        alpha, beta = (1.0, 0.0) if k == 1 else (2.0, -1.0)
        TxNew, out, g = _cheb_step(P, TxA, TxB, post, dinv, W[k], out,
                                   diag_arr, alpha, beta,
                                   emit_g=(k < K - 1))
        TxB, TxA = TxA, TxNew
    return out


def kernel(x, edge_index, batch, lmax, W1, b1, W2, b2, W3, b3,
           fc1_w, fc1_b, fc2_w, fc2_b):
    src = edge_index[0]
    dst = edge_index[1]
    i32 = jnp.int32

    def _bucketize(key, payload):
        order = jnp.argsort(key)
        key_s = key[order]
        pay_s = payload[order]
        b_s = key_s // BSZ
        counts = jnp.bincount(b_s, length=NBUCK)
        padded = ((counts + (BPAD - 1)) // BPAD) * BPAD
        poffs = jnp.concatenate([jnp.zeros((1,), i32),
                                 jnp.cumsum(padded).astype(i32)])
        starts = jnp.concatenate([jnp.zeros((1,), i32),
                                  jnp.cumsum(counts).astype(i32)])
        pos = poffs[b_s] + jnp.arange(E, dtype=i32) - starts[b_s]
        gb = jnp.zeros((TOT_CAP,), i32).at[pos].set(pay_s)
        sb = jnp.full((TOT_CAP,), BSZ, i32).at[pos].set(key_s - b_s * BSZ)
        boff_arr = jnp.pad(poffs[:NBUCK] // 128,
                           (0, 32 - NBUCK)).astype(i32)
        bp_arr = jnp.pad(padded // BPAD, (0, 32 - NBUCK)).astype(i32)
        return (gb.reshape(TOTBLK, 128), sb.reshape(TOTBLK, 128),
                boff_arr, bp_arr)

    dgb, dsb, dboff, dbp = _bucketize(src, dst)
    degF = _sc_spmm(jnp.ones((N, 128), _f32), dgb, dsb, dboff, dbp)

    cc = (2.0 / lmax).astype(_f32)
    cc_arr = cc.reshape(1, 1)
    diag_arr = (cc - 1.0).reshape(1, 1)
    dinv, post = _deg_finalize(degF, cc_arr)

    srcb, dstb, boff_arr, bp_arr = _bucketize(dst, src)

    def spmm(g):
        return _sc_spmm(g, srcb, dstb, boff_arr, bp_arr)

    x_pad = jnp.pad(x, ((0, 0), (0, 14)))
    W1_pad = jnp.pad(W1, ((0, 0), (0, 14), (0, 0)))

    out1 = _cheb_layer(x_pad, None, W1_pad, dinv, post, diag_arr, True, spmm)
    out2 = _cheb_layer(out1, b1, W2, dinv, post, diag_arr, False, spmm)
    out3 = _cheb_layer(out2, b2, W3, dinv, post, diag_arr, False, spmm)

    return _pool_mlp(out3, b3, batch, fc1_w, fc1_b, fc2_w, fc2_b)


# speedup vs baseline: 1.1783x; 1.1783x over previous
import jax
import jax.numpy as jnp
from jax import lax
from jax.experimental import pallas as pl
from jax.experimental.pallas import tpu as pltpu
from jax.experimental.pallas import tpu_sc as plsc

N = 50000
E = 800000
G = 64


NBUCK = 6
BSZ = 8352
B_ACC = 8448
BZR = 528
BPAD = 16384
TOTBLK = 7018
TOT_CAP = TOTBLK * 128

BN = 2000
NB = N // BN

_f32 = jnp.float32


def _sc_spmm(g, srcb, dstb, boff_arr, bp_arr, W):
    mesh = plsc.VectorSubcoreMesh(core_axis_name="c", subcore_axis_name="s")

    def body(g_hbm, srcs, dsts, boff_h, bp_h, out,
             src_v, dst_v, rows, cbuf, zidx, acc, boff_s, bp_s, sem,
             ssem0, ssem1):
        cid = lax.axis_index("c")
        sid = lax.axis_index("s")
        base = lax.broadcasted_iota(jnp.int32, (16,), 0)
        sc_src = rows if W == 128 else cbuf

        pltpu.sync_copy(boff_h, boff_s)
        pltpu.sync_copy(bp_h, bp_s)

        for r in range(5):
            for k in range(8):
                v = base + (sid * BZR + r * 128 + k * 16)
                zidx[r, k * 16:(k + 1) * 16] = jnp.minimum(v, B_ACC - 1)

        zval = jnp.zeros((16,), _f32)

        def zfill(r, _):
            for k in range(W // 16):
                sc_src[r, k * 16:(k + 1) * 16] = zval
            return 0

        def compact(r, _):
            for k in range(W // 16):
                cbuf[r, k * 16:(k + 1) * 16] = rows[r, k * 16:(k + 1) * 16]
            return 0

        for j in range(NBUCK // 2):
            bucket = 2 * j + cid
            boff = boff_s[pl.ds(bucket, 16)][0]
            nc = bp_s[pl.ds(bucket, 16)][0]
            if j > 0:
                plsc.subcore_barrier()
            lax.fori_loop(0, 128, zfill, 0)
            for r in range(5):
                pltpu.sync_copy(sc_src, acc.at[zidx.at[r]])
            tstart = pl.multiple_of(boff + sid * nc * 8, 8)
            plsc.subcore_barrier()

            bufs = [rows, cbuf]
            ssems = [ssem0, ssem1]

            def chunk(m, _):
                moff = pl.multiple_of(tstart + m * 8, 8)
                pltpu.sync_copy(srcs.at[pl.ds(moff, 8)], src_v)
                pltpu.sync_copy(dsts.at[pl.ds(moff, 8)], dst_v)
                sdescs = [None, None]
                for t in range(8):
                    b = t % 2
                    if sdescs[b] is not None:
                        sdescs[b].wait()
                    pltpu.async_copy(g_hbm.at[src_v.at[t]], bufs[b],
                                     sem).wait()
                    sdescs[b] = pltpu.async_copy(
                        bufs[b], acc.at[dst_v.at[t]], ssems[b], add=True)
                for b in range(2):
                    if sdescs[b] is not None:
                        sdescs[b].wait()
                return 0

            lax.fori_loop(0, nc, chunk, 0)
            plsc.subcore_barrier()
            rows_p = jnp.minimum(BSZ, N - bucket * BSZ)
            for c in range(5):
                wstart = pl.multiple_of(
                    jnp.minimum(sid * BZR + c * 128, rows_p - 128), 8)
                gstart = pl.multiple_of(bucket * BSZ + wstart, 8)
                pltpu.sync_copy(acc.at[pl.ds(wstart, 128)], sc_src)
                pltpu.sync_copy(sc_src, out.at[pl.ds(gstart, 128)])

    fn = pl.kernel(
        body,
        out_type=jax.ShapeDtypeStruct((N, W), _f32),
        mesh=mesh,
        scratch_types=[
            pltpu.VMEM((8, 128), jnp.int32),
            pltpu.VMEM((8, 128), jnp.int32),
            pltpu.VMEM((128, 128), _f32),
            pltpu.VMEM((128, 128), _f32),
            pltpu.VMEM((5, 128), jnp.int32),
            pltpu.VMEM_SHARED((B_ACC, W), _f32),
            pltpu.VMEM((32,), jnp.int32),
            pltpu.VMEM((32,), jnp.int32),
            pltpu.SemaphoreType.DMA,
            pltpu.SemaphoreType.DMA,
            pltpu.SemaphoreType.DMA,
        ],
    )
    return fn(g, srcb, dstb, boff_arr, bp_arr)


def _deg_finalize(degF, cc_arr):
    def body(deg_ref, cc_ref, dinv_ref, post_ref):
        d = deg_ref[:, 0:1]
        dinv = jnp.where(d > 0, lax.rsqrt(jnp.maximum(d, 1.0)), 0.0)
        dinv_ref[...] = dinv
        post_ref[...] = (-cc_ref[0, 0]) * dinv

    return pl.pallas_call(
        body,
        grid=(NB,),
        in_specs=[
            pl.BlockSpec((BN, 128), lambda i: (i, 0)),
            pl.BlockSpec(memory_space=pltpu.SMEM),
        ],
        out_specs=[
            pl.BlockSpec((BN, 1), lambda i: (i, 0)),
            pl.BlockSpec((BN, 1), lambda i: (i, 0)),
        ],
        out_shape=[
            jax.ShapeDtypeStruct((N, 1), _f32),
            jax.ShapeDtypeStruct((N, 1), _f32),
        ],
    )(degF, cc_arr)


def _layer_init(z, b, W0, dinv, first):
    Fz = z.shape[1]
    Fout = W0.shape[1]

    def body(*refs):
        if first:
            z_ref, W_ref, dinv_ref, h_ref, out_ref, g_ref = refs
        else:
            z_ref, b_ref, W_ref, dinv_ref, h_ref, out_ref, g_ref = refs
        h = z_ref[...]
        if not first:
            h = jnp.maximum(h + b_ref[...][None, :], 0.0)
        h_ref[...] = h
        out_ref[...] = jnp.dot(h, W_ref[...], preferred_element_type=_f32)
        gv = dinv_ref[...] * h
        if Fz < 128:
            gv = jnp.concatenate(
                [gv, jnp.zeros((BN, 128 - Fz), _f32)], axis=1)
        g_ref[...] = gv

    in_specs = [pl.BlockSpec((BN, Fz), lambda i: (i, 0))]
    args = [z]
    if not first:
        in_specs.append(pl.BlockSpec((Fz,), lambda i: (0,)))
        args.append(b)
    in_specs += [
        pl.BlockSpec((Fz, Fout), lambda i: (0, 0)),
        pl.BlockSpec((BN, 1), lambda i: (i, 0)),
    ]
    args += [W0, dinv]
    return pl.pallas_call(
        body,
        grid=(NB,),
        in_specs=in_specs,
        out_specs=[
            pl.BlockSpec((BN, Fz), lambda i: (i, 0)),
            pl.BlockSpec((BN, Fout), lambda i: (i, 0)),
            pl.BlockSpec((BN, 128), lambda i: (i, 0)),
        ],
        out_shape=[
            jax.ShapeDtypeStruct((N, Fz), _f32),
            jax.ShapeDtypeStruct((N, Fout), _f32),
            jax.ShapeDtypeStruct((N, 128), _f32),
        ],
    )(*args)


def _cheb_step(P, TxA, TxB, post, dinv, Wk, out_in, diag_arr,
               alpha, beta, emit_g):
    F = TxA.shape[1]
    Fout = Wk.shape[1]
    Pw = P.shape[1]

    def body(*refs):
        (P_ref, TxA_ref, TxB_ref, post_ref, dinv_ref, W_ref, oin_ref,
         diag_ref) = refs[:8]
        outs = refs[8:]
        if emit_g:
            TxN_ref, oout_ref, g_ref = outs
        else:
            TxN_ref, oout_ref = outs
        dg = diag_ref[0, 0]
        S = P_ref[...][:, :F]
        t = alpha * (post_ref[...] * S + dg * TxA_ref[...])
        if beta:
            t = t + beta * TxB_ref[...]
        TxN_ref[...] = t
        if emit_g:
            gv = dinv_ref[...] * t
            if F < 128:
                gv = jnp.concatenate(
                    [gv, jnp.zeros((BN, 128 - F), _f32)], axis=1)
            g_ref[...] = gv
        oout_ref[...] = oin_ref[...] + jnp.dot(
            t, W_ref[...], preferred_element_type=_f32)

    out_specs = [
        pl.BlockSpec((BN, F), lambda i: (i, 0)),
        pl.BlockSpec((BN, Fout), lambda i: (i, 0)),
    ]
    out_shape = [
        jax.ShapeDtypeStruct((N, F), _f32),
        jax.ShapeDtypeStruct((N, Fout), _f32),
    ]
    if emit_g:
        out_specs.append(pl.BlockSpec((BN, 128), lambda i: (i, 0)))
        out_shape.append(jax.ShapeDtypeStruct((N, 128), _f32))
    res = pl.pallas_call(
        body,
        grid=(NB,),
        in_specs=[
            pl.BlockSpec((BN, Pw), lambda i: (i, 0)),
            pl.BlockSpec((BN, F), lambda i: (i, 0)),
            pl.BlockSpec((BN, F), lambda i: (i, 0)),
            pl.BlockSpec((BN, 1), lambda i: (i, 0)),
            pl.BlockSpec((BN, 1), lambda i: (i, 0)),
            pl.BlockSpec((F, Fout), lambda i: (0, 0)),
            pl.BlockSpec((BN, Fout), lambda i: (i, 0)),
            pl.BlockSpec(memory_space=pltpu.SMEM),
        ],
        out_specs=out_specs,
        out_shape=out_shape,
    )(P, TxA, TxB, post, dinv, Wk, out_in, diag_arr)
    if emit_g:
        return res
    return res[0], res[1], None


def _pool_mlp(out3, b3, batch, fc1_w, fc1_b, fc2_w, fc2_b):
    def body(h_ref, b3_ref, bat_ref, w1_ref, b1_ref, w2_ref, b2_ref,
             out_ref, sums, cnts):
        i = pl.program_id(0)

        @pl.when(i == 0)
        def _():
            sums[...] = jnp.zeros_like(sums)
            cnts[...] = jnp.zeros_like(cnts)

        h = jnp.maximum(h_ref[...] + b3_ref[...][None, :], 0.0)
        seg = lax.broadcasted_iota(jnp.int32, (G, BN), 0)
        onehot = jnp.where(seg == bat_ref[0], 1.0, 0.0)
        sums[...] += jnp.dot(onehot, h, preferred_element_type=_f32)
        cnts[...] += jnp.sum(onehot, axis=1, keepdims=True)

        @pl.when(i == NB - 1)
        def _():
            pooled = sums[...] / jnp.maximum(cnts[...], 1.0)
            h2 = jnp.maximum(
                jnp.dot(pooled, w1_ref[...], preferred_element_type=_f32)
                + b1_ref[...][None, :], 0.0)
            logits = jnp.dot(h2, w2_ref[...], preferred_element_type=_f32) \
                + b2_ref[...][None, :]
            m = jnp.max(logits, axis=1, keepdims=True)
            lse = jnp.log(jnp.sum(jnp.exp(logits - m), axis=1,
                                  keepdims=True)) + m
            out_ref[...] = logits - lse

    return pl.pallas_call(
        body,
        grid=(NB,),
        in_specs=[
            pl.BlockSpec((BN, 128), lambda i: (i, 0)),
            pl.BlockSpec((128,), lambda i: (0,)),
            pl.BlockSpec((1, 1, BN), lambda i: (i, 0, 0)),
            pl.BlockSpec((128, 32), lambda i: (0, 0)),
            pl.BlockSpec((32,), lambda i: (0,)),
            pl.BlockSpec((32, 10), lambda i: (0, 0)),
            pl.BlockSpec((10,), lambda i: (0,)),
        ],
        out_specs=pl.BlockSpec((G, 10), lambda i: (0, 0)),
        out_shape=jax.ShapeDtypeStruct((G, 10), _f32),
        scratch_shapes=[
            pltpu.VMEM((G, 128), _f32),
            pltpu.VMEM((G, 128), _f32),
        ],
    )(out3, b3, batch.reshape(NB, 1, BN), fc1_w, fc1_b, fc2_w, fc2_b)


def _cheb_layer(h0, bias, W, dinv, post, diag_arr, first, spmm):
    K = W.shape[0]
    TxA, out, g = _layer_init(h0, bias, W[0], dinv, first)
    TxB = TxA
    for k in range(1, K):
        P = spmm(g)
        alpha, beta = (1.0, 0.0) if k == 1 else (2.0, -1.0)
        TxNew, out, g = _cheb_step(P, TxA, TxB, post, dinv, W[k], out,
                                   diag_arr, alpha, beta,
                                   emit_g=(k < K - 1))
        TxB, TxA = TxA, TxNew
    return out


def kernel(x, edge_index, batch, lmax, W1, b1, W2, b2, W3, b3,
           fc1_w, fc1_b, fc2_w, fc2_b):
    src = edge_index[0]
    dst = edge_index[1]
    i32 = jnp.int32

    def _bucketize(key, payload):
        order = jnp.argsort(key)
        key_s = key[order]
        pay_s = payload[order]
        b_s = key_s // BSZ
        counts = jnp.bincount(b_s, length=NBUCK)
        padded = ((counts + (BPAD - 1)) // BPAD) * BPAD
        poffs = jnp.concatenate([jnp.zeros((1,), i32),
                                 jnp.cumsum(padded).astype(i32)])
        starts = jnp.concatenate([jnp.zeros((1,), i32),
                                  jnp.cumsum(counts).astype(i32)])
        pos = poffs[b_s] + jnp.arange(E, dtype=i32) - starts[b_s]
        gb = jnp.zeros((TOT_CAP,), i32).at[pos].set(pay_s)
        sb = jnp.full((TOT_CAP,), BSZ, i32).at[pos].set(key_s - b_s * BSZ)
        boff_arr = jnp.pad(poffs[:NBUCK] // 128,
                           (0, 32 - NBUCK)).astype(i32)
        bp_arr = jnp.pad(padded // BPAD, (0, 32 - NBUCK)).astype(i32)
        return (gb.reshape(TOTBLK, 128), sb.reshape(TOTBLK, 128),
                boff_arr, bp_arr)

    dgb, dsb, dboff, dbp = _bucketize(src, dst)
    degF = _sc_spmm(jnp.ones((N, 128), _f32), dgb, dsb, dboff, dbp, 128)

    cc = (2.0 / lmax).astype(_f32)
    cc_arr = cc.reshape(1, 1)
    diag_arr = (cc - 1.0).reshape(1, 1)
    dinv, post = _deg_finalize(degF, cc_arr)

    srcb, dstb, boff_arr, bp_arr = _bucketize(dst, src)

    def spmm_w(W):
        return lambda g: _sc_spmm(g, srcb, dstb, boff_arr, bp_arr, W)

    x_pad = jnp.pad(x, ((0, 0), (0, 14)))
    W1_pad = jnp.pad(W1, ((0, 0), (0, 14), (0, 0)))

    out1 = _cheb_layer(x_pad, None, W1_pad, dinv, post, diag_arr, True,
                       spmm_w(128))
    out2 = _cheb_layer(out1, b1, W2, dinv, post, diag_arr, False,
                       spmm_w(128))
    out3 = _cheb_layer(out2, b2, W3, dinv, post, diag_arr, False,
                       spmm_w(128))

    return _pool_mlp(out3, b3, batch, fc1_w, fc1_b, fc2_w, fc2_b)
